# baseline (speedup 1.0000x reference)
import functools

import jax
import jax.numpy as jnp
from jax import lax
from jax.experimental import pallas as pl
from jax.experimental.pallas import tpu as pltpu
from jax.experimental.pallas import tpu_sc as plsc

N_NODES = 10000
N_EDGES = 320000
D = 128

NC = 2
NS = 16
NW = NC * NS

EPR = 128
STAGE_ROWS = 32
STAGES_SC0 = 1
STAGES_SC1 = 4
N_ROWS = NS * STAGE_ROWS * (STAGES_SC0 + STAGES_SC1)
E_PAD = N_ROWS * EPR
SC1_BASE = NS * STAGE_ROWS * STAGES_SC0
DUMMY_DST = 10008
N_ACC = 10112
ROWS_PER_TILE_Z = N_ACC // NS
ROWS_PER_TILE_O = 624


def _build_agg():
    mesh = plsc.VectorSubcoreMesh(core_axis_name="c", subcore_axis_name="s")

    @functools.partial(
        pl.kernel,
        mesh=mesh,
        out_type=jax.ShapeDtypeStruct((NC, N_NODES, D), jnp.float32),
        scratch_types=[
            pltpu.VMEM((STAGE_ROWS, EPR), jnp.int32),
            pltpu.VMEM((STAGE_ROWS, EPR), jnp.int32),
            pltpu.VMEM((EPR, D), jnp.float32),
            pltpu.VMEM((EPR, D), jnp.float32),
            pltpu.VMEM_SHARED((N_ACC, D), jnp.float32),
            pltpu.SemaphoreType.DMA,
            pltpu.SemaphoreType.DMA,
        ],
    )
    def agg(src_hbm, dst_hbm, x_hbm, z128_hbm, sum_out,
            src_v, dst_v, msgs0_v, msgs1_v, acc_s, sem0, sem1):
        cid = lax.axis_index("c")
        sid = lax.axis_index("s")

        zb = sid * ROWS_PER_TILE_Z
        pltpu.sync_copy(z128_hbm.at[pl.ds(zb, ROWS_PER_TILE_Z)],
                        acc_s.at[pl.ds(zb, ROWS_PER_TILE_Z)])
        plsc.subcore_barrier()

        nh = STAGE_ROWS // 2

        def run_stage(eb):
            pltpu.sync_copy(src_hbm.at[pl.ds(eb, STAGE_ROWS)], src_v)
            pltpu.sync_copy(dst_hbm.at[pl.ds(eb, STAGE_ROWS)], dst_v)
            pltpu.async_copy(x_hbm.at[src_v.at[0]], msgs0_v, sem0)

            def body(i, carry):
                j0 = 2 * i
                j1 = j0 + 1
                pltpu.async_copy(x_hbm.at[src_v.at[j1]], msgs1_v, sem1)
                pltpu.make_async_copy(x_hbm.at[src_v.at[j0]], msgs0_v,
                                      sem0).wait()
                pltpu.sync_copy(msgs0_v, acc_s.at[dst_v.at[j0]], add=True)

                @pl.when(i < nh - 1)
                def _():
                    pltpu.async_copy(x_hbm.at[src_v.at[j0 + 2]], msgs0_v, sem0)

                pltpu.make_async_copy(x_hbm.at[src_v.at[j1]], msgs1_v,
                                      sem1).wait()
                pltpu.sync_copy(msgs1_v, acc_s.at[dst_v.at[j1]], add=True)
                return carry

            lax.fori_loop(0, nh, body, 0)

        @pl.when(cid == 0)
        def _():
            for st in range(STAGES_SC0):
                run_stage(sid * (STAGES_SC0 * STAGE_ROWS) + st * STAGE_ROWS)

        @pl.when(cid == 1)
        def _():
            for st in range(STAGES_SC1):
                run_stage(SC1_BASE + sid * (STAGES_SC1 * STAGE_ROWS)
                          + st * STAGE_ROWS)

        plsc.subcore_barrier()

        ob = sid * ROWS_PER_TILE_O

        @pl.when(sid < NS - 1)
        def _():
            pltpu.sync_copy(acc_s.at[pl.ds(ob, ROWS_PER_TILE_O)],
                            sum_out.at[cid, pl.ds(ob, ROWS_PER_TILE_O)])

        @pl.when(sid == NS - 1)
        def _():
            tb = (NS - 1) * ROWS_PER_TILE_O
            tail = N_NODES - tb
            pltpu.sync_copy(acc_s.at[pl.ds(tb, tail)],
                            sum_out.at[cid, pl.ds(tb, tail)])

    return agg


def _build_cnt():
    mesh = plsc.VectorSubcoreMesh(core_axis_name="c", subcore_axis_name="s")

    @functools.partial(
        pl.kernel,
        mesh=mesh,
        out_type=jax.ShapeDtypeStruct((NC, N_NODES, D), jnp.float32),
        scratch_types=[
            pltpu.VMEM((STAGE_ROWS, EPR), jnp.int32),
            pltpu.VMEM((EPR, D), jnp.float32),
            pltpu.VMEM_SHARED((N_ACC, D), jnp.float32),
        ],
    )
    def cnt(dst_hbm, ones_hbm, z128_hbm, cnt_out, dst_v, ones_v, cnt_s):
        cid = lax.axis_index("c")
        sid = lax.axis_index("s")

        zb = sid * ROWS_PER_TILE_Z
        pltpu.sync_copy(z128_hbm.at[pl.ds(zb, ROWS_PER_TILE_Z)],
                        cnt_s.at[pl.ds(zb, ROWS_PER_TILE_Z)])
        pltpu.sync_copy(ones_hbm, ones_v)
        plsc.subcore_barrier()

        def run_stage(eb):
            pltpu.sync_copy(dst_hbm.at[pl.ds(eb, STAGE_ROWS)], dst_v)

            def body(j, carry):
                pltpu.sync_copy(ones_v, cnt_s.at[dst_v.at[j]], add=True)
                return carry

            lax.fori_loop(0, STAGE_ROWS, body, 0)

        @pl.when(cid == 0)
        def _():
            for st in range(STAGES_SC0):
                run_stage(sid * (STAGES_SC0 * STAGE_ROWS) + st * STAGE_ROWS)

        @pl.when(cid == 1)
        def _():
            for st in range(STAGES_SC1):
                run_stage(SC1_BASE + sid * (STAGES_SC1 * STAGE_ROWS)
                          + st * STAGE_ROWS)

        plsc.subcore_barrier()

        ob = sid * ROWS_PER_TILE_O

        @pl.when(sid < NS - 1)
        def _():
            pltpu.sync_copy(cnt_s.at[pl.ds(ob, ROWS_PER_TILE_O)],
                            cnt_out.at[cid, pl.ds(ob, ROWS_PER_TILE_O)])

        @pl.when(sid == NS - 1)
        def _():
            tb = (NS - 1) * ROWS_PER_TILE_O
            tail = N_NODES - tb
            pltpu.sync_copy(cnt_s.at[pl.ds(tb, tail)],
                            cnt_out.at[cid, pl.ds(tb, tail)])

    return cnt


_agg = _build_agg()
_cnt = _build_cnt()

_BR = 1000


def _dense1_body(sum_ref, cnt_ref, x_ref, wl_ref, bl_ref, wr_ref, out_ref):
    s = sum_ref[0] + sum_ref[1]
    c = cnt_ref[0][:, :1] + cnt_ref[1][:, :1]
    mean = s / jnp.maximum(c, 1.0)
    h = mean @ wl_ref[...].T + bl_ref[...] + x_ref[...] @ wr_ref[...].T
    out_ref[...] = jnp.tanh(h)


def _dense1(sum_parts, cnt_parts, x, Wl, bl, Wr):
    return pl.pallas_call(
        _dense1_body,
        grid=(N_NODES // _BR,),
        in_specs=[
            pl.BlockSpec((NC, _BR, D), lambda i: (0, i, 0)),
            pl.BlockSpec((NC, _BR, D), lambda i: (0, i, 0)),
            pl.BlockSpec((_BR, D), lambda i: (i, 0)),
            pl.BlockSpec((D, D), lambda i: (0, 0)),
            pl.BlockSpec((1, D), lambda i: (0, 0)),
            pl.BlockSpec((D, D), lambda i: (0, 0)),
        ],
        out_specs=pl.BlockSpec((_BR, D), lambda i: (i, 0)),
        out_shape=jax.ShapeDtypeStruct((N_NODES, D), jnp.float32),
    )(sum_parts, cnt_parts, x, Wl, bl, Wr)


def _dense2_body(sum_ref, cnt_ref, h_ref, wl_ref, bl_ref, wr_ref,
                 wo_ref, bo_ref, out_ref):
    s = sum_ref[0] + sum_ref[1]
    c = cnt_ref[0][:, :1] + cnt_ref[1][:, :1]
    mean = s / jnp.maximum(c, 1.0)
    h2 = mean @ wl_ref[...].T + bl_ref[...] + h_ref[...] @ wr_ref[...].T
    logits = h2 @ wo_ref[...].T + bo_ref[...]
    out_ref[...] = jax.nn.softmax(logits, axis=1)


def _dense2(sum_parts, cnt_parts, h, Wl, bl, Wr, Wout, bout):
    return pl.pallas_call(
        _dense2_body,
        grid=(N_NODES // _BR,),
        in_specs=[
            pl.BlockSpec((NC, _BR, D), lambda i: (0, i, 0)),
            pl.BlockSpec((NC, _BR, D), lambda i: (0, i, 0)),
            pl.BlockSpec((_BR, D), lambda i: (i, 0)),
            pl.BlockSpec((D, D), lambda i: (0, 0)),
            pl.BlockSpec((1, D), lambda i: (0, 0)),
            pl.BlockSpec((D, D), lambda i: (0, 0)),
            pl.BlockSpec((64, D), lambda i: (0, 0)),
            pl.BlockSpec((1, 64), lambda i: (0, 0)),
        ],
        out_specs=pl.BlockSpec((_BR, 64), lambda i: (i, 0)),
        out_shape=jax.ShapeDtypeStruct((N_NODES, 64), jnp.float32),
    )(sum_parts, cnt_parts, h, Wl, bl, Wr, Wout, bout)


def kernel(x, edge_index, Wl1, bl1, Wr1, Wl2, bl2, Wr2, Wout, bout):
    src = edge_index[0].astype(jnp.int32)
    dst = edge_index[1].astype(jnp.int32)
    pad = E_PAD - N_EDGES
    src_p = jnp.concatenate([src, jnp.zeros((pad,), jnp.int32)]).reshape(N_ROWS, EPR)
    dst_p = jnp.concatenate([dst, jnp.full((pad,), DUMMY_DST, jnp.int32)]).reshape(N_ROWS, EPR)
    z128 = jnp.zeros((N_ACC, D), jnp.float32)
    bl1_2d = bl1.reshape(1, D)
    bl2_2d = bl2.reshape(1, D)
    bout_2d = bout.reshape(1, 64)

    ones_rows = jnp.ones((EPR, D), jnp.float32)
    cnt1 = _cnt(dst_p, ones_rows, z128)
    sum1 = _agg(src_p, dst_p, x, z128)
    h1 = _dense1(sum1, cnt1, x, Wl1, bl1_2d, Wr1)
    sum2 = _agg(src_p, dst_p, h1, z128)
    out = _dense2(sum2, cnt1, h1, Wl2, bl2_2d, Wr2, Wout, bout_2d)
    return out


# speedup vs baseline: 1.1470x; 1.1470x over previous
import functools

import jax
import jax.numpy as jnp
from jax import lax
from jax.experimental import pallas as pl
from jax.experimental.pallas import tpu as pltpu
from jax.experimental.pallas import tpu_sc as plsc

N_NODES = 10000
N_EDGES = 320000
D = 128

NC = 2
NS = 16
NW = NC * NS

EPR = 128
STAGE_ROWS = 32
STAGES_SC0 = 4
STAGES_SC1 = 1
N_ROWS = NS * STAGE_ROWS * (STAGES_SC0 + STAGES_SC1)
E_PAD = N_ROWS * EPR
SC1_BASE = NS * STAGE_ROWS * STAGES_SC0
DUMMY_DST = 10008
N_ACC = 10112
ROWS_PER_TILE_Z = N_ACC // NS
ROWS_PER_TILE_O = 624


def _build_agg():
    mesh = plsc.VectorSubcoreMesh(core_axis_name="c", subcore_axis_name="s")

    @functools.partial(
        pl.kernel,
        mesh=mesh,
        out_type=jax.ShapeDtypeStruct((NC, N_NODES, D), jnp.float32),
        scratch_types=[
            pltpu.VMEM((STAGE_ROWS, EPR), jnp.int32),
            pltpu.VMEM((STAGE_ROWS, EPR), jnp.int32),
            pltpu.VMEM((EPR, D), jnp.float32),
            pltpu.VMEM((EPR, D), jnp.float32),
            pltpu.VMEM_SHARED((N_ACC, D), jnp.float32),
            pltpu.SemaphoreType.DMA,
            pltpu.SemaphoreType.DMA,
            pltpu.SemaphoreType.DMA,
            pltpu.SemaphoreType.DMA,
        ],
    )
    def agg(src_hbm, dst_hbm, x_hbm, z128_hbm, sum_out,
            src_v, dst_v, msgs0_v, msgs1_v, acc_s, sem0, sem1, sem2, sem3):
        cid = lax.axis_index("c")
        sid = lax.axis_index("s")

        zb = sid * ROWS_PER_TILE_Z
        pltpu.sync_copy(z128_hbm.at[pl.ds(zb, ROWS_PER_TILE_Z)],
                        acc_s.at[pl.ds(zb, ROWS_PER_TILE_Z)])
        plsc.subcore_barrier()

        nh = STAGE_ROWS // 2

        def run_stage(eb):
            pltpu.sync_copy(src_hbm.at[pl.ds(eb, STAGE_ROWS)], src_v)
            pltpu.sync_copy(dst_hbm.at[pl.ds(eb, STAGE_ROWS)], dst_v)
            pltpu.async_copy(x_hbm.at[src_v.at[0]], msgs0_v, sem0)
            pltpu.async_copy(x_hbm.at[src_v.at[1]], msgs1_v, sem1)

            def body(i, carry):
                j0 = 2 * i
                j1 = j0 + 1
                pltpu.make_async_copy(x_hbm.at[src_v.at[j0]], msgs0_v,
                                      sem0).wait()
                s0 = pltpu.async_copy(msgs0_v, acc_s.at[dst_v.at[j0]], sem2,
                                      add=True)
                pltpu.make_async_copy(x_hbm.at[src_v.at[j1]], msgs1_v,
                                      sem1).wait()
                s1 = pltpu.async_copy(msgs1_v, acc_s.at[dst_v.at[j1]], sem3,
                                      add=True)
                s0.wait()

                @pl.when(i < nh - 1)
                def _():
                    pltpu.async_copy(x_hbm.at[src_v.at[j0 + 2]], msgs0_v, sem0)

                s1.wait()

                @pl.when(i < nh - 1)
                def _():
                    pltpu.async_copy(x_hbm.at[src_v.at[j1 + 2]], msgs1_v, sem1)

                return carry

            lax.fori_loop(0, nh, body, 0)

        @pl.when(cid == 0)
        def _():
            for st in range(STAGES_SC0):
                run_stage(sid * (STAGES_SC0 * STAGE_ROWS) + st * STAGE_ROWS)

        @pl.when(cid == 1)
        def _():
            for st in range(STAGES_SC1):
                run_stage(SC1_BASE + sid * (STAGES_SC1 * STAGE_ROWS)
                          + st * STAGE_ROWS)

        plsc.subcore_barrier()

        ob = sid * ROWS_PER_TILE_O

        @pl.when(sid < NS - 1)
        def _():
            pltpu.sync_copy(acc_s.at[pl.ds(ob, ROWS_PER_TILE_O)],
                            sum_out.at[cid, pl.ds(ob, ROWS_PER_TILE_O)])

        @pl.when(sid == NS - 1)
        def _():
            tb = (NS - 1) * ROWS_PER_TILE_O
            tail = N_NODES - tb
            pltpu.sync_copy(acc_s.at[pl.ds(tb, tail)],
                            sum_out.at[cid, pl.ds(tb, tail)])

    return agg


def _build_cnt():
    mesh = plsc.VectorSubcoreMesh(core_axis_name="c", subcore_axis_name="s")

    @functools.partial(
        pl.kernel,
        mesh=mesh,
        out_type=jax.ShapeDtypeStruct((NC, N_NODES, D), jnp.float32),
        scratch_types=[
            pltpu.VMEM((STAGE_ROWS, EPR), jnp.int32),
            pltpu.VMEM((EPR, D), jnp.float32),
            pltpu.VMEM_SHARED((N_ACC, D), jnp.float32),
        ],
    )
    def cnt(dst_hbm, ones_hbm, z128_hbm, cnt_out, dst_v, ones_v, cnt_s):
        cid = lax.axis_index("c")
        sid = lax.axis_index("s")

        zb = sid * ROWS_PER_TILE_Z
        pltpu.sync_copy(z128_hbm.at[pl.ds(zb, ROWS_PER_TILE_Z)],
                        cnt_s.at[pl.ds(zb, ROWS_PER_TILE_Z)])
        pltpu.sync_copy(ones_hbm, ones_v)
        plsc.subcore_barrier()

        def run_stage(eb):
            pltpu.sync_copy(dst_hbm.at[pl.ds(eb, STAGE_ROWS)], dst_v)

            def body(j, carry):
                pltpu.sync_copy(ones_v, cnt_s.at[dst_v.at[j]], add=True)
                return carry

            lax.fori_loop(0, STAGE_ROWS, body, 0)

        @pl.when(cid == 0)
        def _():
            for st in range(STAGES_SC0):
                run_stage(sid * (STAGES_SC0 * STAGE_ROWS) + st * STAGE_ROWS)

        @pl.when(cid == 1)
        def _():
            for st in range(STAGES_SC1):
                run_stage(SC1_BASE + sid * (STAGES_SC1 * STAGE_ROWS)
                          + st * STAGE_ROWS)

        plsc.subcore_barrier()

        ob = sid * ROWS_PER_TILE_O

        @pl.when(sid < NS - 1)
        def _():
            pltpu.sync_copy(cnt_s.at[pl.ds(ob, ROWS_PER_TILE_O)],
                            cnt_out.at[cid, pl.ds(ob, ROWS_PER_TILE_O)])

        @pl.when(sid == NS - 1)
        def _():
            tb = (NS - 1) * ROWS_PER_TILE_O
            tail = N_NODES - tb
            pltpu.sync_copy(cnt_s.at[pl.ds(tb, tail)],
                            cnt_out.at[cid, pl.ds(tb, tail)])

    return cnt


_agg = _build_agg()
_cnt = _build_cnt()

_BR = 1000


def _dense1_body(sum_ref, cnt_ref, x_ref, wl_ref, bl_ref, wr_ref, out_ref):
    s = sum_ref[0] + sum_ref[1]
    c = cnt_ref[0][:, :1] + cnt_ref[1][:, :1]
    mean = s / jnp.maximum(c, 1.0)
    h = mean @ wl_ref[...].T + bl_ref[...] + x_ref[...] @ wr_ref[...].T
    out_ref[...] = jnp.tanh(h)


def _dense1(sum_parts, cnt_parts, x, Wl, bl, Wr):
    return pl.pallas_call(
        _dense1_body,
        grid=(N_NODES // _BR,),
        in_specs=[
            pl.BlockSpec((NC, _BR, D), lambda i: (0, i, 0)),
            pl.BlockSpec((NC, _BR, D), lambda i: (0, i, 0)),
            pl.BlockSpec((_BR, D), lambda i: (i, 0)),
            pl.BlockSpec((D, D), lambda i: (0, 0)),
            pl.BlockSpec((1, D), lambda i: (0, 0)),
            pl.BlockSpec((D, D), lambda i: (0, 0)),
        ],
        out_specs=pl.BlockSpec((_BR, D), lambda i: (i, 0)),
        out_shape=jax.ShapeDtypeStruct((N_NODES, D), jnp.float32),
    )(sum_parts, cnt_parts, x, Wl, bl, Wr)


def _dense2_body(sum_ref, cnt_ref, h_ref, wl_ref, bl_ref, wr_ref,
                 wo_ref, bo_ref, out_ref):
    s = sum_ref[0] + sum_ref[1]
    c = cnt_ref[0][:, :1] + cnt_ref[1][:, :1]
    mean = s / jnp.maximum(c, 1.0)
    h2 = mean @ wl_ref[...].T + bl_ref[...] + h_ref[...] @ wr_ref[...].T
    logits = h2 @ wo_ref[...].T + bo_ref[...]
    out_ref[...] = jax.nn.softmax(logits, axis=1)


def _dense2(sum_parts, cnt_parts, h, Wl, bl, Wr, Wout, bout):
    return pl.pallas_call(
        _dense2_body,
        grid=(N_NODES // _BR,),
        in_specs=[
            pl.BlockSpec((NC, _BR, D), lambda i: (0, i, 0)),
            pl.BlockSpec((NC, _BR, D), lambda i: (0, i, 0)),
            pl.BlockSpec((_BR, D), lambda i: (i, 0)),
            pl.BlockSpec((D, D), lambda i: (0, 0)),
            pl.BlockSpec((1, D), lambda i: (0, 0)),
            pl.BlockSpec((D, D), lambda i: (0, 0)),
            pl.BlockSpec((64, D), lambda i: (0, 0)),
            pl.BlockSpec((1, 64), lambda i: (0, 0)),
        ],
        out_specs=pl.BlockSpec((_BR, 64), lambda i: (i, 0)),
        out_shape=jax.ShapeDtypeStruct((N_NODES, 64), jnp.float32),
    )(sum_parts, cnt_parts, h, Wl, bl, Wr, Wout, bout)


def kernel(x, edge_index, Wl1, bl1, Wr1, Wl2, bl2, Wr2, Wout, bout):
    src = edge_index[0].astype(jnp.int32)
    dst = edge_index[1].astype(jnp.int32)
    pad = E_PAD - N_EDGES
    src_p = jnp.concatenate([src, jnp.zeros((pad,), jnp.int32)]).reshape(N_ROWS, EPR)
    dst_p = jnp.concatenate([dst, jnp.full((pad,), DUMMY_DST, jnp.int32)]).reshape(N_ROWS, EPR)
    z128 = jnp.zeros((N_ACC, D), jnp.float32)
    bl1_2d = bl1.reshape(1, D)
    bl2_2d = bl2.reshape(1, D)
    bout_2d = bout.reshape(1, 64)

    ones_rows = jnp.ones((EPR, D), jnp.float32)
    cnt1 = _cnt(dst_p, ones_rows, z128)
    sum1 = _agg(src_p, dst_p, x, z128)
    h1 = _dense1(sum1, cnt1, x, Wl1, bl1_2d, Wr1)
    sum2 = _agg(src_p, dst_p, h1, z128)
    out = _dense2(sum2, cnt1, h1, Wl2, bl2_2d, Wr2, Wout, bout_2d)
    return out


# speedup vs baseline: 1.1478x; 1.0007x over previous
import functools

import jax
import jax.numpy as jnp
from jax import lax
from jax.experimental import pallas as pl
from jax.experimental.pallas import tpu as pltpu
from jax.experimental.pallas import tpu_sc as plsc

N_NODES = 10000
N_EDGES = 320000
D = 128

NC = 2
NS = 16
NW = NC * NS

EPR = 128
STAGE_ROWS = 32
STAGES_SC0 = 4
STAGES_SC1 = 1
N_ROWS = NS * STAGE_ROWS * (STAGES_SC0 + STAGES_SC1)
E_PAD = N_ROWS * EPR
SC1_BASE = NS * STAGE_ROWS * STAGES_SC0
DUMMY_DST = 10008
N_ACC = 10112
ROWS_PER_TILE_Z = N_ACC // NS
ROWS_PER_TILE_O = 624


def _build_agg():
    mesh = plsc.VectorSubcoreMesh(core_axis_name="c", subcore_axis_name="s")

    @functools.partial(
        pl.kernel,
        mesh=mesh,
        out_type=jax.ShapeDtypeStruct((NC, N_NODES, D), jnp.float32),
        scratch_types=[
            pltpu.VMEM((STAGE_ROWS, EPR), jnp.int32),
            pltpu.VMEM((STAGE_ROWS, EPR), jnp.int32),
            pltpu.VMEM((EPR, D), jnp.float32),
            pltpu.VMEM((EPR, D), jnp.float32),
            pltpu.VMEM_SHARED((N_ACC, D), jnp.float32),
            pltpu.SemaphoreType.DMA,
            pltpu.SemaphoreType.DMA,
            pltpu.SemaphoreType.DMA,
            pltpu.SemaphoreType.DMA,
        ],
    )
    def agg(src_hbm, dst_hbm, x_hbm, z128_hbm, sum_out,
            src_v, dst_v, msgs0_v, msgs1_v, acc_s, sem0, sem1, sem2, sem3):
        cid = lax.axis_index("c")
        sid = lax.axis_index("s")

        zb = sid * ROWS_PER_TILE_Z
        pltpu.sync_copy(z128_hbm.at[pl.ds(zb, ROWS_PER_TILE_Z)],
                        acc_s.at[pl.ds(zb, ROWS_PER_TILE_Z)])
        plsc.subcore_barrier()

        nh = STAGE_ROWS // 2

        def run_stage(eb):
            pltpu.sync_copy(src_hbm.at[pl.ds(eb, STAGE_ROWS)], src_v)
            pltpu.sync_copy(dst_hbm.at[pl.ds(eb, STAGE_ROWS)], dst_v)
            pltpu.async_copy(x_hbm.at[src_v.at[0]], msgs0_v, sem0)
            pltpu.async_copy(x_hbm.at[src_v.at[1]], msgs1_v, sem1)

            def body(i, carry):
                j0 = 2 * i
                j1 = j0 + 1
                pltpu.make_async_copy(x_hbm.at[src_v.at[j0]], msgs0_v,
                                      sem0).wait()
                s0 = pltpu.async_copy(msgs0_v, acc_s.at[dst_v.at[j0]], sem2,
                                      add=True)
                pltpu.make_async_copy(x_hbm.at[src_v.at[j1]], msgs1_v,
                                      sem1).wait()
                s1 = pltpu.async_copy(msgs1_v, acc_s.at[dst_v.at[j1]], sem3,
                                      add=True)
                s0.wait()

                @pl.when(i < nh - 1)
                def _():
                    pltpu.async_copy(x_hbm.at[src_v.at[j0 + 2]], msgs0_v, sem0)

                s1.wait()

                @pl.when(i < nh - 1)
                def _():
                    pltpu.async_copy(x_hbm.at[src_v.at[j1 + 2]], msgs1_v, sem1)

                return carry

            lax.fori_loop(0, nh, body, 0)

        @pl.when(cid == 0)
        def _():
            for st in range(STAGES_SC0):
                run_stage(sid * (STAGES_SC0 * STAGE_ROWS) + st * STAGE_ROWS)

        @pl.when(cid == 1)
        def _():
            for st in range(STAGES_SC1):
                run_stage(SC1_BASE + sid * (STAGES_SC1 * STAGE_ROWS)
                          + st * STAGE_ROWS)

        plsc.subcore_barrier()

        ob = sid * ROWS_PER_TILE_O

        @pl.when(sid < NS - 1)
        def _():
            pltpu.sync_copy(acc_s.at[pl.ds(ob, ROWS_PER_TILE_O)],
                            sum_out.at[cid, pl.ds(ob, ROWS_PER_TILE_O)])

        @pl.when(sid == NS - 1)
        def _():
            tb = (NS - 1) * ROWS_PER_TILE_O
            tail = N_NODES - tb
            pltpu.sync_copy(acc_s.at[pl.ds(tb, tail)],
                            sum_out.at[cid, pl.ds(tb, tail)])

    return agg


def _build_cnt():
    mesh = plsc.VectorSubcoreMesh(core_axis_name="c", subcore_axis_name="s")

    @functools.partial(
        pl.kernel,
        mesh=mesh,
        out_type=jax.ShapeDtypeStruct((NC, N_NODES, D), jnp.float32),
        scratch_types=[
            pltpu.VMEM((STAGE_ROWS, EPR), jnp.int32),
            pltpu.VMEM((EPR, D), jnp.float32),
            pltpu.VMEM_SHARED((N_ACC, D), jnp.float32),
            pltpu.SemaphoreType.DMA,
        ],
    )
    def cnt(dst_hbm, ones_hbm, z128_hbm, cnt_out, dst_v, ones_v, cnt_s, semc):
        cid = lax.axis_index("c")
        sid = lax.axis_index("s")

        zb = sid * ROWS_PER_TILE_Z
        pltpu.sync_copy(z128_hbm.at[pl.ds(zb, ROWS_PER_TILE_Z)],
                        cnt_s.at[pl.ds(zb, ROWS_PER_TILE_Z)])
        pltpu.sync_copy(ones_hbm, ones_v)
        plsc.subcore_barrier()

        def run_stage(eb):
            pltpu.sync_copy(dst_hbm.at[pl.ds(eb, STAGE_ROWS)], dst_v)

            def body(j, carry):
                pltpu.async_copy(ones_v, cnt_s.at[dst_v.at[j]], semc,
                                 add=True)
                return carry

            lax.fori_loop(0, STAGE_ROWS, body, 0)

            def drain(j, carry):
                pltpu.make_async_copy(ones_hbm, ones_v, semc).wait()
                return carry

            lax.fori_loop(0, STAGE_ROWS, drain, 0)

        @pl.when(cid == 0)
        def _():
            for st in range(STAGES_SC0):
                run_stage(sid * (STAGES_SC0 * STAGE_ROWS) + st * STAGE_ROWS)

        @pl.when(cid == 1)
        def _():
            for st in range(STAGES_SC1):
                run_stage(SC1_BASE + sid * (STAGES_SC1 * STAGE_ROWS)
                          + st * STAGE_ROWS)

        plsc.subcore_barrier()

        ob = sid * ROWS_PER_TILE_O

        @pl.when(sid < NS - 1)
        def _():
            pltpu.sync_copy(cnt_s.at[pl.ds(ob, ROWS_PER_TILE_O)],
                            cnt_out.at[cid, pl.ds(ob, ROWS_PER_TILE_O)])

        @pl.when(sid == NS - 1)
        def _():
            tb = (NS - 1) * ROWS_PER_TILE_O
            tail = N_NODES - tb
            pltpu.sync_copy(cnt_s.at[pl.ds(tb, tail)],
                            cnt_out.at[cid, pl.ds(tb, tail)])

    return cnt


_agg = _build_agg()
_cnt = _build_cnt()

_BR = 1000


def _dense1_body(sum_ref, cnt_ref, x_ref, wl_ref, bl_ref, wr_ref, out_ref):
    s = sum_ref[0] + sum_ref[1]
    c = cnt_ref[0][:, :1] + cnt_ref[1][:, :1]
    mean = s / jnp.maximum(c, 1.0)
    h = mean @ wl_ref[...].T + bl_ref[...] + x_ref[...] @ wr_ref[...].T
    out_ref[...] = jnp.tanh(h)


def _dense1(sum_parts, cnt_parts, x, Wl, bl, Wr):
    return pl.pallas_call(
        _dense1_body,
        grid=(N_NODES // _BR,),
        in_specs=[
            pl.BlockSpec((NC, _BR, D), lambda i: (0, i, 0)),
            pl.BlockSpec((NC, _BR, D), lambda i: (0, i, 0)),
            pl.BlockSpec((_BR, D), lambda i: (i, 0)),
            pl.BlockSpec((D, D), lambda i: (0, 0)),
            pl.BlockSpec((1, D), lambda i: (0, 0)),
            pl.BlockSpec((D, D), lambda i: (0, 0)),
        ],
        out_specs=pl.BlockSpec((_BR, D), lambda i: (i, 0)),
        out_shape=jax.ShapeDtypeStruct((N_NODES, D), jnp.float32),
    )(sum_parts, cnt_parts, x, Wl, bl, Wr)


def _dense2_body(sum_ref, cnt_ref, h_ref, wl_ref, bl_ref, wr_ref,
                 wo_ref, bo_ref, out_ref):
    s = sum_ref[0] + sum_ref[1]
    c = cnt_ref[0][:, :1] + cnt_ref[1][:, :1]
    mean = s / jnp.maximum(c, 1.0)
    h2 = mean @ wl_ref[...].T + bl_ref[...] + h_ref[...] @ wr_ref[...].T
    logits = h2 @ wo_ref[...].T + bo_ref[...]
    out_ref[...] = jax.nn.softmax(logits, axis=1)


def _dense2(sum_parts, cnt_parts, h, Wl, bl, Wr, Wout, bout):
    return pl.pallas_call(
        _dense2_body,
        grid=(N_NODES // _BR,),
        in_specs=[
            pl.BlockSpec((NC, _BR, D), lambda i: (0, i, 0)),
            pl.BlockSpec((NC, _BR, D), lambda i: (0, i, 0)),
            pl.BlockSpec((_BR, D), lambda i: (i, 0)),
            pl.BlockSpec((D, D), lambda i: (0, 0)),
            pl.BlockSpec((1, D), lambda i: (0, 0)),
            pl.BlockSpec((D, D), lambda i: (0, 0)),
            pl.BlockSpec((64, D), lambda i: (0, 0)),
            pl.BlockSpec((1, 64), lambda i: (0, 0)),
        ],
        out_specs=pl.BlockSpec((_BR, 64), lambda i: (i, 0)),
        out_shape=jax.ShapeDtypeStruct((N_NODES, 64), jnp.float32),
    )(sum_parts, cnt_parts, h, Wl, bl, Wr, Wout, bout)


def kernel(x, edge_index, Wl1, bl1, Wr1, Wl2, bl2, Wr2, Wout, bout):
    src = edge_index[0].astype(jnp.int32)
    dst = edge_index[1].astype(jnp.int32)
    pad = E_PAD - N_EDGES
    src_p = jnp.concatenate([src, jnp.zeros((pad,), jnp.int32)]).reshape(N_ROWS, EPR)
    dst_p = jnp.concatenate([dst, jnp.full((pad,), DUMMY_DST, jnp.int32)]).reshape(N_ROWS, EPR)
    z128 = jnp.zeros((N_ACC, D), jnp.float32)
    bl1_2d = bl1.reshape(1, D)
    bl2_2d = bl2.reshape(1, D)
    bout_2d = bout.reshape(1, 64)

    ones_rows = jnp.ones((EPR, D), jnp.float32)
    cnt1 = _cnt(dst_p, ones_rows, z128)
    sum1 = _agg(src_p, dst_p, x, z128)
    h1 = _dense1(sum1, cnt1, x, Wl1, bl1_2d, Wr1)
    sum2 = _agg(src_p, dst_p, h1, z128)
    out = _dense2(sum2, cnt1, h1, Wl2, bl2_2d, Wr2, Wout, bout_2d)
    return out


# speedup vs baseline: 1.1492x; 1.0013x over previous
import functools

import jax
import jax.numpy as jnp
from jax import lax
from jax.experimental import pallas as pl
from jax.experimental.pallas import tpu as pltpu
from jax.experimental.pallas import tpu_sc as plsc

N_NODES = 10000
N_EDGES = 320000
D = 128

NC = 2
NS = 16
NW = NC * NS

EPR = 128
STAGE_ROWS = 64
ROWS_SC0 = 128
ROWS_SC1 = 32
N_ROWS = NS * (ROWS_SC0 + ROWS_SC1)
E_PAD = N_ROWS * EPR
SC1_BASE = NS * ROWS_SC0
DUMMY_DST = 10008
N_ACC = 10112
ROWS_PER_TILE_Z = N_ACC // NS
ROWS_PER_TILE_O = 624


def _build_agg():
    mesh = plsc.VectorSubcoreMesh(core_axis_name="c", subcore_axis_name="s")

    @functools.partial(
        pl.kernel,
        mesh=mesh,
        out_type=jax.ShapeDtypeStruct((NC, N_NODES, D), jnp.float32),
        scratch_types=[
            pltpu.VMEM((STAGE_ROWS, EPR), jnp.int32),
            pltpu.VMEM((STAGE_ROWS, EPR), jnp.int32),
            pltpu.VMEM((EPR, D), jnp.float32),
            pltpu.VMEM((EPR, D), jnp.float32),
            pltpu.VMEM_SHARED((N_ACC, D), jnp.float32),
            pltpu.SemaphoreType.DMA,
            pltpu.SemaphoreType.DMA,
            pltpu.SemaphoreType.DMA,
            pltpu.SemaphoreType.DMA,
        ],
    )
    def agg(src_hbm, dst_hbm, x_hbm, z128_hbm, sum_out,
            src_v, dst_v, msgs0_v, msgs1_v, acc_s, sem0, sem1, sem2, sem3):
        cid = lax.axis_index("c")
        sid = lax.axis_index("s")

        zb = sid * ROWS_PER_TILE_Z
        pltpu.sync_copy(z128_hbm.at[pl.ds(zb, ROWS_PER_TILE_Z)],
                        acc_s.at[pl.ds(zb, ROWS_PER_TILE_Z)])
        plsc.subcore_barrier()

        def run_stage(eb, rows):
            nh = rows // 2
            pltpu.sync_copy(src_hbm.at[pl.ds(eb, rows)],
                            src_v.at[pl.ds(0, rows)])
            pltpu.sync_copy(dst_hbm.at[pl.ds(eb, rows)],
                            dst_v.at[pl.ds(0, rows)])
            pltpu.async_copy(x_hbm.at[src_v.at[0]], msgs0_v, sem0)
            pltpu.async_copy(x_hbm.at[src_v.at[1]], msgs1_v, sem1)

            def body(i, carry):
                j0 = 2 * i
                j1 = j0 + 1
                pltpu.make_async_copy(x_hbm.at[src_v.at[j0]], msgs0_v,
                                      sem0).wait()
                s0 = pltpu.async_copy(msgs0_v, acc_s.at[dst_v.at[j0]], sem2,
                                      add=True)
                pltpu.make_async_copy(x_hbm.at[src_v.at[j1]], msgs1_v,
                                      sem1).wait()
                s1 = pltpu.async_copy(msgs1_v, acc_s.at[dst_v.at[j1]], sem3,
                                      add=True)
                s0.wait()

                @pl.when(i < nh - 1)
                def _():
                    pltpu.async_copy(x_hbm.at[src_v.at[j0 + 2]], msgs0_v, sem0)

                s1.wait()

                @pl.when(i < nh - 1)
                def _():
                    pltpu.async_copy(x_hbm.at[src_v.at[j1 + 2]], msgs1_v, sem1)

                return carry

            lax.fori_loop(0, nh, body, 0)

        @pl.when(cid == 0)
        def _():
            for st in range(ROWS_SC0 // STAGE_ROWS):
                run_stage(sid * ROWS_SC0 + st * STAGE_ROWS, STAGE_ROWS)

        @pl.when(cid == 1)
        def _():
            run_stage(SC1_BASE + sid * ROWS_SC1, ROWS_SC1)

        plsc.subcore_barrier()

        ob = sid * ROWS_PER_TILE_O

        @pl.when(sid < NS - 1)
        def _():
            pltpu.sync_copy(acc_s.at[pl.ds(ob, ROWS_PER_TILE_O)],
                            sum_out.at[cid, pl.ds(ob, ROWS_PER_TILE_O)])

        @pl.when(sid == NS - 1)
        def _():
            tb = (NS - 1) * ROWS_PER_TILE_O
            tail = N_NODES - tb
            pltpu.sync_copy(acc_s.at[pl.ds(tb, tail)],
                            sum_out.at[cid, pl.ds(tb, tail)])

    return agg


def _build_cnt():
    mesh = plsc.VectorSubcoreMesh(core_axis_name="c", subcore_axis_name="s")

    @functools.partial(
        pl.kernel,
        mesh=mesh,
        out_type=jax.ShapeDtypeStruct((NC, N_NODES, D), jnp.float32),
        scratch_types=[
            pltpu.VMEM((STAGE_ROWS, EPR), jnp.int32),
            pltpu.VMEM((EPR, D), jnp.float32),
            pltpu.VMEM_SHARED((N_ACC, D), jnp.float32),
            pltpu.SemaphoreType.DMA,
        ],
    )
    def cnt(dst_hbm, ones_hbm, z128_hbm, cnt_out, dst_v, ones_v, cnt_s, semc):
        cid = lax.axis_index("c")
        sid = lax.axis_index("s")

        zb = sid * ROWS_PER_TILE_Z
        pltpu.sync_copy(z128_hbm.at[pl.ds(zb, ROWS_PER_TILE_Z)],
                        cnt_s.at[pl.ds(zb, ROWS_PER_TILE_Z)])
        pltpu.sync_copy(ones_hbm, ones_v)
        plsc.subcore_barrier()

        def run_stage(eb, rows):
            pltpu.sync_copy(dst_hbm.at[pl.ds(eb, rows)],
                            dst_v.at[pl.ds(0, rows)])

            def body(j, carry):
                pltpu.async_copy(ones_v, cnt_s.at[dst_v.at[j]], semc,
                                 add=True)
                return carry

            lax.fori_loop(0, rows, body, 0)

            def drain(j, carry):
                pltpu.make_async_copy(ones_hbm, ones_v, semc).wait()
                return carry

            lax.fori_loop(0, rows, drain, 0)

        @pl.when(cid == 0)
        def _():
            for st in range(ROWS_SC0 // STAGE_ROWS):
                run_stage(sid * ROWS_SC0 + st * STAGE_ROWS, STAGE_ROWS)

        @pl.when(cid == 1)
        def _():
            run_stage(SC1_BASE + sid * ROWS_SC1, ROWS_SC1)

        plsc.subcore_barrier()

        ob = sid * ROWS_PER_TILE_O

        @pl.when(sid < NS - 1)
        def _():
            pltpu.sync_copy(cnt_s.at[pl.ds(ob, ROWS_PER_TILE_O)],
                            cnt_out.at[cid, pl.ds(ob, ROWS_PER_TILE_O)])

        @pl.when(sid == NS - 1)
        def _():
            tb = (NS - 1) * ROWS_PER_TILE_O
            tail = N_NODES - tb
            pltpu.sync_copy(cnt_s.at[pl.ds(tb, tail)],
                            cnt_out.at[cid, pl.ds(tb, tail)])

    return cnt


_agg = _build_agg()
_cnt = _build_cnt()

_BR = 1000


def _dense1_body(sum_ref, cnt_ref, x_ref, wl_ref, bl_ref, wr_ref, out_ref):
    s = sum_ref[0] + sum_ref[1]
    c = cnt_ref[0][:, :1] + cnt_ref[1][:, :1]
    mean = s / jnp.maximum(c, 1.0)
    h = mean @ wl_ref[...].T + bl_ref[...] + x_ref[...] @ wr_ref[...].T
    out_ref[...] = jnp.tanh(h)


def _dense1(sum_parts, cnt_parts, x, Wl, bl, Wr):
    return pl.pallas_call(
        _dense1_body,
        grid=(N_NODES // _BR,),
        in_specs=[
            pl.BlockSpec((NC, _BR, D), lambda i: (0, i, 0)),
            pl.BlockSpec((NC, _BR, D), lambda i: (0, i, 0)),
            pl.BlockSpec((_BR, D), lambda i: (i, 0)),
            pl.BlockSpec((D, D), lambda i: (0, 0)),
            pl.BlockSpec((1, D), lambda i: (0, 0)),
            pl.BlockSpec((D, D), lambda i: (0, 0)),
        ],
        out_specs=pl.BlockSpec((_BR, D), lambda i: (i, 0)),
        out_shape=jax.ShapeDtypeStruct((N_NODES, D), jnp.float32),
    )(sum_parts, cnt_parts, x, Wl, bl, Wr)


def _dense2_body(sum_ref, cnt_ref, h_ref, wl_ref, bl_ref, wr_ref,
                 wo_ref, bo_ref, out_ref):
    s = sum_ref[0] + sum_ref[1]
    c = cnt_ref[0][:, :1] + cnt_ref[1][:, :1]
    mean = s / jnp.maximum(c, 1.0)
    h2 = mean @ wl_ref[...].T + bl_ref[...] + h_ref[...] @ wr_ref[...].T
    logits = h2 @ wo_ref[...].T + bo_ref[...]
    out_ref[...] = jax.nn.softmax(logits, axis=1)


def _dense2(sum_parts, cnt_parts, h, Wl, bl, Wr, Wout, bout):
    return pl.pallas_call(
        _dense2_body,
        grid=(N_NODES // _BR,),
        in_specs=[
            pl.BlockSpec((NC, _BR, D), lambda i: (0, i, 0)),
            pl.BlockSpec((NC, _BR, D), lambda i: (0, i, 0)),
            pl.BlockSpec((_BR, D), lambda i: (i, 0)),
            pl.BlockSpec((D, D), lambda i: (0, 0)),
            pl.BlockSpec((1, D), lambda i: (0, 0)),
            pl.BlockSpec((D, D), lambda i: (0, 0)),
            pl.BlockSpec((64, D), lambda i: (0, 0)),
            pl.BlockSpec((1, 64), lambda i: (0, 0)),
        ],
        out_specs=pl.BlockSpec((_BR, 64), lambda i: (i, 0)),
        out_shape=jax.ShapeDtypeStruct((N_NODES, 64), jnp.float32),
    )(sum_parts, cnt_parts, h, Wl, bl, Wr, Wout, bout)


def kernel(x, edge_index, Wl1, bl1, Wr1, Wl2, bl2, Wr2, Wout, bout):
    src = edge_index[0].astype(jnp.int32)
    dst = edge_index[1].astype(jnp.int32)
    pad = E_PAD - N_EDGES
    src_p = jnp.concatenate([src, jnp.zeros((pad,), jnp.int32)]).reshape(N_ROWS, EPR)
    dst_p = jnp.concatenate([dst, jnp.full((pad,), DUMMY_DST, jnp.int32)]).reshape(N_ROWS, EPR)
    z128 = jnp.zeros((N_ACC, D), jnp.float32)
    bl1_2d = bl1.reshape(1, D)
    bl2_2d = bl2.reshape(1, D)
    bout_2d = bout.reshape(1, 64)

    ones_rows = jnp.ones((EPR, D), jnp.float32)
    cnt1 = _cnt(dst_p, ones_rows, z128)
    sum1 = _agg(src_p, dst_p, x, z128)
    h1 = _dense1(sum1, cnt1, x, Wl1, bl1_2d, Wr1)
    sum2 = _agg(src_p, dst_p, h1, z128)
    out = _dense2(sum2, cnt1, h1, Wl2, bl2_2d, Wr2, Wout, bout_2d)
    return out
